# baseline (speedup 1.0000x reference)
import jax
import jax.numpy as jnp
from jax import lax
from jax.experimental import pallas as pl
from jax.experimental.pallas import tpu as pltpu
from jax.experimental.pallas import tpu_sc as plsc

B, S, D = 4, 4096, 2048

S_SC = 512
S_TC = S - S_SC
BS = 512

_INFO = plsc.get_sparse_core_info()
NC, NS, L = _INFO.num_cores, _INFO.num_subcores, _INFO.num_lanes
NW = NC * NS
SPW = S_SC // NW
R = 8
NCHUNK = SPW // R
NUNIT = NCHUNK * B
NV = D // L


def _sc_body(x_hbm, tbl_hbm, out_hbm,
             tb0, tb1, xb0, xb1, xb2, xb3,
             semt0, semt1, semx0, semx1, semx2, semx3,
             semo0, semo1, semo2, semo3):
    wid = lax.axis_index("s") * NC + lax.axis_index("c")
    s_base = wid * SPW

    tbufs = (tb0, tb1)
    xbufs = (xb0, xb1, xb2, xb3)
    semts = (semt0, semt1)
    semxs = (semx0, semx1, semx2, semx3)
    semos = (semo0, semo1, semo2, semo3)

    def x_row(j):
        c = j >> 2
        b = j & 3
        return b * S + s_base + c * R

    def o_row(j):
        c = j >> 2
        b = j & 3
        return b * S_SC + s_base + c * R

    def tbl_copy(c, ts):
        return pltpu.make_async_copy(
            tbl_hbm.at[pl.ds(s_base + c * R, R)], tbufs[ts], semts[ts])

    def x_copy(j, slot):
        return pltpu.make_async_copy(
            x_hbm.at[pl.ds(x_row(j), R)], xbufs[slot], semxs[slot])

    def out_copy(j, slot):
        return pltpu.make_async_copy(
            xbufs[slot], out_hbm.at[pl.ds(o_row(j), R)], semos[slot])

    tbl_copy(0, 0).start()
    x_copy(0, 0).start()
    x_copy(1, 1).start()

    def step(t, carry):
        for q in range(8):
            j = 8 * t + q
            b = q & 3
            cpar = (q >> 2) & 1
            slot = q % 4
            c = 2 * t + (q >> 2)

            if b == 0:
                @pl.when(c + 1 < NCHUNK)
                def _():
                    tbl_copy(c + 1, cpar ^ 1).start()

            nslot = (q + 2) % 4

            @pl.when(j >= 2)
            def _():
                out_copy(j - 2, nslot).wait()

            @pl.when(j + 2 < NUNIT)
            def _():
                x_copy(j + 2, nslot).start()

            if b == 0:
                tbl_copy(c, cpar).wait()
            x_copy(j, slot).wait()

            tb = tbufs[cpar]
            xb = xbufs[slot]

            @plsc.parallel_loop(0, R * NV, unroll=4)
            def _(k):
                r = k >> 7
                sl = pl.ds((k & (NV - 1)) * L, L)
                xb.at[r][sl] = xb.at[r][sl] + tb.at[r][sl]

            out_copy(j, slot).start()
        return carry

    lax.fori_loop(0, NUNIT // 8, step, 0)

    out_copy(NUNIT - 2, (NUNIT - 2) % 4).wait()
    out_copy(NUNIT - 1, (NUNIT - 1) % 4).wait()


def _tc_body(x_ref, tbl_ref, o_ref):
    o_ref[...] = x_ref[...] + tbl_ref[...][None, :, :]


def _merge_body(sc_ref, al_ref, o_ref):
    del al_ref
    o_ref[...] = sc_ref[...]


@jax.jit
def kernel(encoded_tokens, position_table):
    x2 = encoded_tokens.reshape(B * S, D)

    run_sc = pl.kernel(
        _sc_body,
        out_type=jax.ShapeDtypeStruct((B * S_SC, D), jnp.float32),
        mesh=plsc.VectorSubcoreMesh(core_axis_name="c", subcore_axis_name="s"),
        scratch_types=[
            pltpu.VMEM((R, D), jnp.float32),
            pltpu.VMEM((R, D), jnp.float32),
            pltpu.VMEM((R, D), jnp.float32),
            pltpu.VMEM((R, D), jnp.float32),
            pltpu.VMEM((R, D), jnp.float32),
            pltpu.VMEM((R, D), jnp.float32),
            pltpu.SemaphoreType.DMA,
            pltpu.SemaphoreType.DMA,
            pltpu.SemaphoreType.DMA,
            pltpu.SemaphoreType.DMA,
            pltpu.SemaphoreType.DMA,
            pltpu.SemaphoreType.DMA,
            pltpu.SemaphoreType.DMA,
            pltpu.SemaphoreType.DMA,
            pltpu.SemaphoreType.DMA,
            pltpu.SemaphoreType.DMA,
        ],
    )
    sc_out = run_sc(x2, position_table).reshape(B, S_SC, D)

    nsc = S_SC // BS
    tc_out = pl.pallas_call(
        _tc_body,
        grid=(S_TC // BS, B),
        in_specs=[
            pl.BlockSpec((1, BS, D), lambda i, b: (b, nsc + i, 0)),
            pl.BlockSpec((BS, D), lambda i, b: (nsc + i, 0)),
        ],
        out_specs=pl.BlockSpec((1, BS, D), lambda i, b: (b, nsc + i, 0)),
        out_shape=jax.ShapeDtypeStruct((B, S, D), jnp.float32),
    )(encoded_tokens, position_table)

    return pl.pallas_call(
        _merge_body,
        grid=(nsc, B),
        in_specs=[
            pl.BlockSpec((1, BS, D), lambda i, b: (b, i, 0)),
            pl.BlockSpec(memory_space=pl.ANY),
        ],
        out_specs=pl.BlockSpec((1, BS, D), lambda i, b: (b, i, 0)),
        out_shape=jax.ShapeDtypeStruct((B, S, D), jnp.float32),
        input_output_aliases={1: 0},
    )(sc_out, tc_out)


# speedup vs baseline: 1.0534x; 1.0534x over previous
import jax
import jax.numpy as jnp
from jax import lax
from jax.experimental import pallas as pl
from jax.experimental.pallas import tpu as pltpu
from jax.experimental.pallas import tpu_sc as plsc

B, S, D = 4, 4096, 2048

S_SC = 1024
S_TC = S - S_SC
BS = 1024

_INFO = plsc.get_sparse_core_info()
NC, NS, L = _INFO.num_cores, _INFO.num_subcores, _INFO.num_lanes
NW = NC * NS
SPW = S_SC // NW
R = 8
NCHUNK = SPW // R
NUNIT = NCHUNK * B
NV = D // L


def _sc_body(x_hbm, tbl_hbm, out_hbm,
             tb0, tb1, xb0, xb1, xb2, xb3,
             semt0, semt1, semx0, semx1, semx2, semx3,
             semo0, semo1, semo2, semo3):
    wid = lax.axis_index("s") * NC + lax.axis_index("c")
    s_base = wid * SPW

    tbufs = (tb0, tb1)
    xbufs = (xb0, xb1, xb2, xb3)
    semts = (semt0, semt1)
    semxs = (semx0, semx1, semx2, semx3)
    semos = (semo0, semo1, semo2, semo3)

    def x_row(j):
        c = j >> 2
        b = j & 3
        return b * S + s_base + c * R

    def o_row(j):
        c = j >> 2
        b = j & 3
        return b * S + s_base + c * R

    def tbl_copy(c, ts):
        return pltpu.make_async_copy(
            tbl_hbm.at[pl.ds(s_base + c * R, R)], tbufs[ts], semts[ts])

    def x_copy(j, slot):
        return pltpu.make_async_copy(
            x_hbm.at[pl.ds(x_row(j), R)], xbufs[slot], semxs[slot])

    def out_copy(j, slot):
        return pltpu.make_async_copy(
            xbufs[slot], out_hbm.at[pl.ds(o_row(j), R)], semos[slot])

    tbl_copy(0, 0).start()
    x_copy(0, 0).start()
    x_copy(1, 1).start()

    def step(t, carry):
        for q in range(8):
            j = 8 * t + q
            b = q & 3
            cpar = (q >> 2) & 1
            slot = q % 4
            c = 2 * t + (q >> 2)

            if b == 0:
                @pl.when(c + 1 < NCHUNK)
                def _():
                    tbl_copy(c + 1, cpar ^ 1).start()

            nslot = (q + 2) % 4

            @pl.when(j >= 2)
            def _():
                out_copy(j - 2, nslot).wait()

            @pl.when(j + 2 < NUNIT)
            def _():
                x_copy(j + 2, nslot).start()

            if b == 0:
                tbl_copy(c, cpar).wait()
            x_copy(j, slot).wait()

            tb = tbufs[cpar]
            xb = xbufs[slot]

            @plsc.parallel_loop(0, R * NV, unroll=4)
            def _(k):
                r = k >> 7
                sl = pl.ds((k & (NV - 1)) * L, L)
                xb.at[r][sl] = xb.at[r][sl] + tb.at[r][sl]

            out_copy(j, slot).start()
        return carry

    lax.fori_loop(0, NUNIT // 8, step, 0)

    out_copy(NUNIT - 2, (NUNIT - 2) % 4).wait()
    out_copy(NUNIT - 1, (NUNIT - 1) % 4).wait()


def _tc_body(x_ref, tbl_ref, al_ref, o_ref):
    del al_ref
    o_ref[...] = x_ref[...] + tbl_ref[...][None, :, :]


@jax.jit
def kernel(encoded_tokens, position_table):
    x2 = encoded_tokens.reshape(B * S, D)

    run_sc = pl.kernel(
        _sc_body,
        out_type=jax.ShapeDtypeStruct((B * S, D), jnp.float32),
        mesh=plsc.VectorSubcoreMesh(core_axis_name="c", subcore_axis_name="s"),
        scratch_types=[
            pltpu.VMEM((R, D), jnp.float32),
            pltpu.VMEM((R, D), jnp.float32),
            pltpu.VMEM((R, D), jnp.float32),
            pltpu.VMEM((R, D), jnp.float32),
            pltpu.VMEM((R, D), jnp.float32),
            pltpu.VMEM((R, D), jnp.float32),
            pltpu.SemaphoreType.DMA,
            pltpu.SemaphoreType.DMA,
            pltpu.SemaphoreType.DMA,
            pltpu.SemaphoreType.DMA,
            pltpu.SemaphoreType.DMA,
            pltpu.SemaphoreType.DMA,
            pltpu.SemaphoreType.DMA,
            pltpu.SemaphoreType.DMA,
            pltpu.SemaphoreType.DMA,
            pltpu.SemaphoreType.DMA,
        ],
    )
    sc_full = run_sc(x2, position_table).reshape(B, S, D)

    nsc = S_SC // BS
    return pl.pallas_call(
        _tc_body,
        grid=(S_TC // BS, B),
        in_specs=[
            pl.BlockSpec((1, BS, D), lambda i, b: (b, nsc + i, 0)),
            pl.BlockSpec((BS, D), lambda i, b: (nsc + i, 0)),
            pl.BlockSpec(memory_space=pl.ANY),
        ],
        out_specs=pl.BlockSpec((1, BS, D), lambda i, b: (b, nsc + i, 0)),
        out_shape=jax.ShapeDtypeStruct((B, S, D), jnp.float32),
        input_output_aliases={2: 0},
    )(encoded_tokens, position_table, sc_full)
